# baseline (speedup 1.0000x reference)
import jax
import jax.numpy as jnp
from jax import lax
from jax.experimental import pallas as pl
from jax.experimental.pallas import tpu as pltpu
from jax.experimental.pallas import tpu_sc as plsc

_N_DOCS = 100000
_BATCH = 4096
_LIST = 50
_NC = 2
_NS = 16
_NW = _NC * _NS
_ROWS_PER_W = _BATCH // _NW
_ELEMS_PER_W = _ROWS_PER_W * _LIST
_GROUPS = _ROWS_PER_W // 16


def _cascade_body(x_hbm, table_hbm, out_hbm,
                  idx_v, idxt_v, vals_v, out_v, shared_tab,
                  sem_i, sem_t, sem_g):
    cid = lax.axis_index("c")
    sid = lax.axis_index("s")
    wid = sid * _NC + cid
    base = wid * _ELEMS_PER_W

    with jax.named_scope("spfill"):
        @pl.when(sid == 0)
        def _fill():
            pltpu.async_copy(table_hbm, shared_tab, sem_t).wait()

    cp_i = pltpu.async_copy(x_hbm.at[pl.ds(base, _ELEMS_PER_W)], idx_v, sem_i)
    cp_i.wait()

    lane50 = lax.iota(jnp.int32, 16) * _LIST
    ones = jnp.ones((16,), jnp.float32)

    with jax.named_scope("tr"):
        def _tr_group(g, _):
            @plsc.parallel_loop(0, _LIST)
            def _tr(i):
                xi = plsc.load_gather(idx_v, [lane50 + (g * 16 * _LIST + i)])
                idxt_v[pl.ds(i * _ROWS_PER_W + g * 16, 16)] = xi
            return 0
        lax.fori_loop(0, _GROUPS, _tr_group, 0)

    with jax.named_scope("bar"):
        plsc.subcore_barrier()

    with jax.named_scope("gather"):
        pltpu.async_copy(shared_tab.at[idxt_v], vals_v, sem_g).wait()

    with jax.named_scope("casc"):
        def _casc_group(g, _):
            @plsc.parallel_loop(0, _LIST, carry=ones)
            def _casc(i, p):
                v = vals_v[pl.ds(i * _ROWS_PER_W + g * 16, 16)]
                r = 1.0 / (1.0 + jnp.exp(-v))
                o = p * r
                plsc.store_scatter(out_v, [lane50 + (g * 16 * _LIST + i)], o)
                return p - o
            return 0
        lax.fori_loop(0, _GROUPS, _casc_group, 0)

    with jax.named_scope("wb"):
        pltpu.sync_copy(out_v, out_hbm.at[pl.ds(base, _ELEMS_PER_W)])


def kernel(x, table):
    xf = x.reshape(_BATCH * _LIST)
    tf = table.reshape(_N_DOCS)
    mesh = plsc.VectorSubcoreMesh(core_axis_name="c", subcore_axis_name="s")
    out = pl.kernel(
        _cascade_body,
        out_type=jax.ShapeDtypeStruct((_BATCH * _LIST,), jnp.float32),
        mesh=mesh,
        compiler_params=pltpu.CompilerParams(needs_layout_passes=False),
        scratch_types=[
            pltpu.VMEM((_ELEMS_PER_W,), jnp.int32),
            pltpu.VMEM((_ELEMS_PER_W,), jnp.int32),
            pltpu.VMEM((_ELEMS_PER_W,), jnp.float32),
            pltpu.VMEM((_ELEMS_PER_W,), jnp.float32),
            pltpu.VMEM_SHARED((_N_DOCS,), jnp.float32),
            pltpu.SemaphoreType.DMA,
            pltpu.SemaphoreType.DMA,
            pltpu.SemaphoreType.DMA,
        ],
    )(xf, tf)
    return out.reshape(_BATCH, _LIST)


# speedup vs baseline: 1.0068x; 1.0068x over previous
import jax
import jax.numpy as jnp
from jax import lax
from jax.experimental import pallas as pl
from jax.experimental.pallas import tpu as pltpu
from jax.experimental.pallas import tpu_sc as plsc

_N_DOCS = 100000
_BATCH = 4096
_LIST = 50
_NC = 2
_NS = 16
_NW = _NC * _NS
_ROWS_PER_W = _BATCH // _NW
_ELEMS_PER_W = _ROWS_PER_W * _LIST
_GROUPS = _ROWS_PER_W // 16


def _cascade_body(x_hbm, table_hbm, out_hbm,
                  idx_v, idxt_v, vals_v, out_v, shared_tab,
                  sem_i, sem_t, sem_g):
    cid = lax.axis_index("c")
    sid = lax.axis_index("s")
    wid = sid * _NC + cid
    base = wid * _ROWS_PER_W

    @pl.when(sid == 0)
    def _fill():
        pltpu.async_copy(table_hbm, shared_tab, sem_t).wait()

    cp_i = pltpu.async_copy(x_hbm.at[pl.ds(base, _ROWS_PER_W)], idx_v, sem_i)
    cp_i.wait()

    lane = lax.iota(jnp.int32, 16)
    zero16 = jnp.zeros((16,), jnp.int32)
    ones = jnp.ones((16,), jnp.float32)

    @plsc.parallel_loop(0, _LIST)
    def _tr(i):
        col = zero16 + i
        for g in range(_GROUPS):
            xi = plsc.load_gather(idx_v, [lane + g * 16, col])
            idxt_v[pl.ds(i * _ROWS_PER_W + g * 16, 16)] = xi

    plsc.subcore_barrier()

    pltpu.async_copy(shared_tab.at[idxt_v], vals_v, sem_g).wait()

    @plsc.parallel_loop(0, _LIST, carry=tuple(ones for _ in range(_GROUPS)))
    def _casc(i, ps):
        col = zero16 + i
        new_ps = []
        for g in range(_GROUPS):
            v = vals_v[pl.ds(i * _ROWS_PER_W + g * 16, 16)]
            r = 1.0 / (1.0 + jnp.exp(-v))
            o = ps[g] * r
            plsc.store_scatter(out_v, [lane + g * 16, col], o)
            new_ps.append(ps[g] - o)
        return tuple(new_ps)

    pltpu.sync_copy(out_v, out_hbm.at[pl.ds(base, _ROWS_PER_W)])


def kernel(x, table):
    tf = table.reshape(_N_DOCS)
    mesh = plsc.VectorSubcoreMesh(core_axis_name="c", subcore_axis_name="s")
    return pl.kernel(
        _cascade_body,
        out_type=jax.ShapeDtypeStruct((_BATCH, _LIST), jnp.float32),
        mesh=mesh,
        compiler_params=pltpu.CompilerParams(
            needs_layout_passes=False, use_tc_tiling_on_sc=True),
        scratch_types=[
            pltpu.VMEM((_ROWS_PER_W, _LIST), jnp.int32),
            pltpu.VMEM((_ELEMS_PER_W,), jnp.int32),
            pltpu.VMEM((_ELEMS_PER_W,), jnp.float32),
            pltpu.VMEM((_ROWS_PER_W, _LIST), jnp.float32),
            pltpu.VMEM_SHARED((_N_DOCS,), jnp.float32),
            pltpu.SemaphoreType.DMA,
            pltpu.SemaphoreType.DMA,
            pltpu.SemaphoreType.DMA,
        ],
    )(x, tf)


# speedup vs baseline: 1.0361x; 1.0291x over previous
import jax
import jax.numpy as jnp
from jax import lax
from jax.experimental import pallas as pl
from jax.experimental.pallas import tpu as pltpu
from jax.experimental.pallas import tpu_sc as plsc

_N_DOCS = 100000
_BATCH = 4096
_LIST = 50
_NC = 2
_NS = 16
_NW = _NC * _NS
_ROWS_PER_W = _BATCH // _NW
_ELEMS_PER_W = _ROWS_PER_W * _LIST
_GROUPS = _ROWS_PER_W // 16


def _cascade_body(x_hbm, table_hbm, out_hbm,
                  idx_v, idxt_v, vals_v, out_v, shared_tab,
                  sem_i, sem_t, sem_g):
    cid = lax.axis_index("c")
    sid = lax.axis_index("s")
    wid = sid * _NC + cid
    base = wid * _ELEMS_PER_W

    with jax.named_scope("spfill"):
        @pl.when(sid == 0)
        def _fill():
            pltpu.async_copy(table_hbm, shared_tab, sem_t).wait()

    cp_i = pltpu.async_copy(x_hbm.at[pl.ds(base, _ELEMS_PER_W)], idx_v, sem_i)
    cp_i.wait()

    lane50 = lax.iota(jnp.int32, 16) * _LIST
    ones = jnp.ones((16,), jnp.float32)

    with jax.named_scope("tr"):
        @plsc.parallel_loop(0, _LIST)
        def _tr(i):
            for g in range(_GROUPS):
                xi = plsc.load_gather(idx_v, [lane50 + (g * 16 * _LIST + i)])
                idxt_v[pl.ds(i * _ROWS_PER_W + g * 16, 16)] = xi

    with jax.named_scope("bar"):
        plsc.subcore_barrier()

    with jax.named_scope("gather"):
        pltpu.async_copy(shared_tab.at[idxt_v], vals_v, sem_g).wait()

    with jax.named_scope("casc"):
        @plsc.parallel_loop(0, _LIST, carry=tuple(ones for _ in range(_GROUPS)))
        def _casc(i, ps):
            new_ps = []
            for g in range(_GROUPS):
                v = vals_v[pl.ds(i * _ROWS_PER_W + g * 16, 16)]
                r = 1.0 / (1.0 + jnp.exp(-v))
                o = ps[g] * r
                plsc.store_scatter(out_v, [lane50 + (g * 16 * _LIST + i)], o)
                new_ps.append(ps[g] - o)
            return tuple(new_ps)

    with jax.named_scope("wb"):
        pltpu.sync_copy(out_v, out_hbm.at[pl.ds(base, _ELEMS_PER_W)])


def kernel(x, table):
    xf = x.reshape(_BATCH * _LIST)
    tf = table.reshape(_N_DOCS)
    mesh = plsc.VectorSubcoreMesh(core_axis_name="c", subcore_axis_name="s")
    out = pl.kernel(
        _cascade_body,
        out_type=jax.ShapeDtypeStruct((_BATCH * _LIST,), jnp.float32),
        mesh=mesh,
        compiler_params=pltpu.CompilerParams(needs_layout_passes=False),
        scratch_types=[
            pltpu.VMEM((_ELEMS_PER_W,), jnp.int32),
            pltpu.VMEM((_ELEMS_PER_W,), jnp.int32),
            pltpu.VMEM((_ELEMS_PER_W,), jnp.float32),
            pltpu.VMEM((_ELEMS_PER_W,), jnp.float32),
            pltpu.VMEM_SHARED((_N_DOCS,), jnp.float32),
            pltpu.SemaphoreType.DMA,
            pltpu.SemaphoreType.DMA,
            pltpu.SemaphoreType.DMA,
        ],
    )(xf, tf)
    return out.reshape(_BATCH, _LIST)


# speedup vs baseline: 1.0618x; 1.0248x over previous
import jax
import jax.numpy as jnp
from jax import lax
from jax.experimental import pallas as pl
from jax.experimental.pallas import tpu as pltpu
from jax.experimental.pallas import tpu_sc as plsc

_N_DOCS = 100000
_BATCH = 4096
_LIST = 50
_NC = 2
_NS = 16
_NW = _NC * _NS
_ROWS_PER_W = _BATCH // _NW
_ELEMS_PER_W = _ROWS_PER_W * _LIST
_GROUPS = _ROWS_PER_W // 16


def _cascade_body(x_hbm, table_hbm, out_hbm,
                  idx_v, idxt_v, vals_v, out_v, shared_tab,
                  sem_i, sem_t, sem_g):
    cid = lax.axis_index("c")
    sid = lax.axis_index("s")
    wid = sid * _NC + cid
    base = wid * _ELEMS_PER_W

    cp_i = pltpu.async_copy(x_hbm.at[pl.ds(base, _ELEMS_PER_W)], idx_v, sem_i)

    with jax.named_scope("spfill"):
        @pl.when(sid == 0)
        def _fill():
            pltpu.async_copy(table_hbm, shared_tab, sem_t).wait()

    cp_i.wait()

    lane50 = lax.iota(jnp.int32, 16) * _LIST
    ones = jnp.ones((16,), jnp.float32)

    with jax.named_scope("tr"):
        @plsc.parallel_loop(0, _LIST)
        def _tr(i):
            for g in range(_GROUPS):
                xi = plsc.load_gather(idx_v, [lane50 + (g * 16 * _LIST + i)])
                idxt_v[pl.ds(i * _ROWS_PER_W + g * 16, 16)] = xi

    with jax.named_scope("bar"):
        plsc.subcore_barrier()

    with jax.named_scope("gather"):
        pltpu.async_copy(shared_tab.at[idxt_v], vals_v, sem_g).wait()

    with jax.named_scope("casc"):
        @plsc.parallel_loop(0, _LIST, carry=tuple(ones for _ in range(_GROUPS)))
        def _casc(i, ps):
            new_ps = []
            for g in range(_GROUPS):
                v = vals_v[pl.ds(i * _ROWS_PER_W + g * 16, 16)]
                r = 1.0 / (1.0 + jnp.exp(-v))
                o = ps[g] * r
                plsc.store_scatter(out_v, [lane50 + (g * 16 * _LIST + i)], o)
                new_ps.append(ps[g] - o)
            return tuple(new_ps)

    with jax.named_scope("wb"):
        pltpu.sync_copy(out_v, out_hbm.at[pl.ds(base, _ELEMS_PER_W)])


def kernel(x, table):
    xf = x.reshape(_BATCH * _LIST)
    tf = table.reshape(_N_DOCS)
    mesh = plsc.VectorSubcoreMesh(core_axis_name="c", subcore_axis_name="s")
    out = pl.kernel(
        _cascade_body,
        out_type=jax.ShapeDtypeStruct((_BATCH * _LIST,), jnp.float32),
        mesh=mesh,
        compiler_params=pltpu.CompilerParams(needs_layout_passes=False),
        scratch_types=[
            pltpu.VMEM((_ELEMS_PER_W,), jnp.int32),
            pltpu.VMEM((_ELEMS_PER_W,), jnp.int32),
            pltpu.VMEM((_ELEMS_PER_W,), jnp.float32),
            pltpu.VMEM((_ELEMS_PER_W,), jnp.float32),
            pltpu.VMEM_SHARED((_N_DOCS,), jnp.float32),
            pltpu.SemaphoreType.DMA,
            pltpu.SemaphoreType.DMA,
            pltpu.SemaphoreType.DMA,
        ],
    )(xf, tf)
    return out.reshape(_BATCH, _LIST)
